# baseline (speedup 1.0000x reference)
import jax
import jax.numpy as jnp
from jax import lax
from jax.experimental import pallas as pl
from jax.experimental.pallas import tpu as pltpu
from jax.experimental.pallas import tpu_sc as plsc

N = 10000
D = 128
E = 320000

NC = 2
NS = 16
NW = NC * NS

CHUNK = 80
EPT = 10240
CPT = EPT // CHUNK
E_PAD = NW * EPT
DUMMY = N
ACC_ROWS = NW * 320
ZROWS = ACC_ROWS // NS
NB = 4
NWIN = CPT // NB
SHIFT = 14
MASK = (1 << SHIFT) - 1

_mesh = plsc.VectorSubcoreMesh(
    core_axis_name="c", subcore_axis_name="s", num_cores=NC, num_subcores=NS
)


def _unpack(pkb, b, src_b, dst_b):
    for k in range(CHUNK // 16):
        v = pkb[b, pl.ds(k * 16, 16)]
        src_b[pl.ds(k * 16, 16)] = jnp.bitwise_and(v, MASK)
        dst_b[pl.ds(k * 16, 16)] = lax.shift_right_logical(v, SHIFT)


def _deg_unpack_dst(pkb, b, dst_b):
    for k in range(CHUNK // 16):
        v = pkb[b, pl.ds(k * 16, 16)]
        dst_b[pl.ds(k * 16, 16)] = lax.shift_right_logical(v, SHIFT)


def _deg_body(pk_hbm, zeros_hbm, out_hbm,
              pkb0, pkb1, db0, db1, db2, db3, ones_v,
              p0, p1, s0, s1, s2, s3, deg_sh):
    pkb = (pkb0, pkb1)
    psem = (p0, p1)
    dbuf = (db0, db1, db2, db3)
    ssem = (s0, s1, s2, s3)
    cid = lax.axis_index("c")
    sid = lax.axis_index("s")
    wid = cid * NS + sid

    def pk_fetch(w, half):
        pltpu.async_copy(pk_hbm.at[wid, w], pkb[half], psem[half])

    def pk_wait(half):
        pltpu.make_async_copy(pk_hbm.at[wid, 0], pkb[half], psem[half]).wait()

    def scatter(b):
        pltpu.async_copy(ones_v, deg_sh.at[dbuf[b]], ssem[b], add=True)

    def scatter_wait(b):
        pltpu.make_async_copy(ones_v, deg_sh.at[dbuf[b]], ssem[b]).wait()

    pltpu.sync_copy(zeros_hbm, deg_sh.at[pl.ds(sid * ZROWS, ZROWS)])
    pk_fetch(0, 0)
    pk_fetch(1, 1)
    for k in range(CHUNK // 16):
        ones_v[pl.ds(k * 16, 16)] = jnp.ones((16,), jnp.float32)
    plsc.subcore_barrier()
    pk_wait(0)
    for b in range(NB):
        _deg_unpack_dst(pkb0, b, dbuf[b])
        scatter(b)

    def macro(i, carry):
        w2 = 2 * i + 2
        pk_fetch(w2, 0)
        pk_wait(1)
        for b in range(NB):
            scatter_wait(b)
            _deg_unpack_dst(pkb1, b, dbuf[b])
            scatter(b)
        pk_fetch(w2 + 1, 1)
        pk_wait(0)
        for b in range(NB):
            scatter_wait(b)
            _deg_unpack_dst(pkb0, b, dbuf[b])
            scatter(b)
        return carry

    lax.fori_loop(0, NWIN // 2 - 1, macro, 0)
    pk_wait(1)
    for b in range(NB):
        scatter_wait(b)
        _deg_unpack_dst(pkb1, b, dbuf[b])
        scatter(b)
    for b in range(NB):
        scatter_wait(b)
    plsc.subcore_barrier()
    pltpu.sync_copy(
        deg_sh.at[pl.ds(sid * ZROWS, ZROWS)],
        out_hbm.at[cid, pl.ds(sid * ZROWS, ZROWS)],
    )


_deg_kernel = pl.kernel(
    _deg_body,
    out_type=jax.ShapeDtypeStruct((NC, ACC_ROWS), jnp.float32),
    mesh=_mesh,
    scratch_types=(
        [pltpu.VMEM((NB, CHUNK), jnp.int32)] * 2
        + [pltpu.VMEM((CHUNK,), jnp.int32)] * NB
        + [pltpu.VMEM((CHUNK,), jnp.float32)]
        + [pltpu.SemaphoreType.DMA] * (2 + NB)
        + [pltpu.VMEM_SHARED((ACC_ROWS,), jnp.float32)]
    ),
)


def _scatter_body(xs_hbm, pk_hbm, zeros_hbm, out_hbm,
                  pkb0, pkb1, sb0, sb1, sb2, sb3, db0, db1, db2, db3,
                  r0, r1, r2, r3,
                  p0, p1, g0, g1, g2, g3, s0, s1, s2, s3, acc_sh):
    pkb = (pkb0, pkb1)
    psem = (p0, p1)
    sbuf = (sb0, sb1, sb2, sb3)
    dbuf = (db0, db1, db2, db3)
    rows = (r0, r1, r2, r3)
    gsem = (g0, g1, g2, g3)
    ssem = (s0, s1, s2, s3)
    cid = lax.axis_index("c")
    sid = lax.axis_index("s")
    wid = cid * NS + sid

    def pk_fetch(w, half):
        pltpu.async_copy(pk_hbm.at[wid, w], pkb[half], psem[half])

    def pk_wait(half):
        pltpu.make_async_copy(pk_hbm.at[wid, 0], pkb[half], psem[half]).wait()

    def gather(b):
        pltpu.async_copy(xs_hbm.at[sbuf[b]], rows[b], gsem[b])

    def gather_wait(b):
        pltpu.make_async_copy(xs_hbm.at[sbuf[b]], rows[b], gsem[b]).wait()

    def scatter(b):
        pltpu.async_copy(rows[b], acc_sh.at[dbuf[b]], ssem[b], add=True)

    def scatter_wait(b):
        pltpu.make_async_copy(rows[b], acc_sh.at[dbuf[b]], ssem[b]).wait()

    pltpu.sync_copy(zeros_hbm, acc_sh.at[pl.ds(sid * ZROWS, ZROWS)])
    pk_fetch(0, 0)
    pk_fetch(1, 1)
    plsc.subcore_barrier()
    pk_wait(0)
    for b in range(NB):
        _unpack(pkb0, b, sbuf[b], dbuf[b])
        gather(b)

    def macro(i, carry):
        w2 = 2 * i + 2
        pk_fetch(w2, 0)
        for b in range(NB):
            gather_wait(b)
            scatter(b)
        pk_wait(1)
        for b in range(NB):
            scatter_wait(b)
            _unpack(pkb1, b, sbuf[b], dbuf[b])
            gather(b)
        pk_fetch(w2 + 1, 1)
        for b in range(NB):
            gather_wait(b)
            scatter(b)
        pk_wait(0)
        for b in range(NB):
            scatter_wait(b)
            _unpack(pkb0, b, sbuf[b], dbuf[b])
            gather(b)
        return carry

    lax.fori_loop(0, NWIN // 2 - 1, macro, 0)
    for b in range(NB):
        gather_wait(b)
        scatter(b)
    pk_wait(1)
    for b in range(NB):
        scatter_wait(b)
        _unpack(pkb1, b, sbuf[b], dbuf[b])
        gather(b)
    for b in range(NB):
        gather_wait(b)
        scatter(b)
    for b in range(NB):
        scatter_wait(b)
    plsc.subcore_barrier()
    pltpu.sync_copy(
        acc_sh.at[pl.ds(sid * ZROWS, ZROWS)],
        out_hbm.at[cid, pl.ds(sid * ZROWS, ZROWS)],
    )


_scatter_kernel = pl.kernel(
    _scatter_body,
    out_type=jax.ShapeDtypeStruct((NC, ACC_ROWS, D), jnp.float32),
    mesh=_mesh,
    scratch_types=(
        [pltpu.VMEM((NB, CHUNK), jnp.int32)] * 2
        + [pltpu.VMEM((CHUNK,), jnp.int32)] * (2 * NB)
        + [pltpu.VMEM((CHUNK, D), jnp.float32)] * NB
        + [pltpu.SemaphoreType.DMA] * (2 + 2 * NB)
        + [pltpu.VMEM_SHARED((ACC_ROWS, D), jnp.float32)]
    ),
)


BR = 2000


def _dinv(degp_ref):
    deg = degp_ref[:, 0:1] + degp_ref[:, 1:2] + 1.0
    return lax.rsqrt(deg)


def _tc_mm_body(x_ref, w_ref, mm_ref):
    mm_ref[...] = jnp.dot(x_ref[...], w_ref[...], preferred_element_type=jnp.float32)


def _tc_scale_body(degp_ref, mm_ref, xs_ref):
    xs_ref[...] = mm_ref[...] * _dinv(degp_ref)


def _tc_b_body(degp_ref, s_ref, xs_ref, b_ref, w_ref, out_ref):
    dv = _dinv(degp_ref)
    h = s_ref[0] + s_ref[1] + xs_ref[...]
    h = jnp.maximum(dv * h + b_ref[...], 0.0)
    out_ref[...] = (
        jnp.dot(h, w_ref[...], preferred_element_type=jnp.float32) * dv
    )


def _tc_c_body(degp_ref, q_ref, xs_ref, b_ref, out_ref):
    dv = _dinv(degp_ref)
    out_ref[...] = dv * (q_ref[0] + q_ref[1] + xs_ref[...]) + b_ref[...]


def _tc_mm(x, W1):
    return pl.pallas_call(
        _tc_mm_body,
        grid=(N // BR,),
        in_specs=[
            pl.BlockSpec((BR, D), lambda i: (i, 0)),
            pl.BlockSpec((D, D), lambda i: (0, 0)),
        ],
        out_specs=pl.BlockSpec((BR, D), lambda i: (i, 0)),
        out_shape=jax.ShapeDtypeStruct((N, D), jnp.float32),
    )(x, W1)


def _tc_scale(degp, mm):
    return pl.pallas_call(
        _tc_scale_body,
        grid=(N // BR,),
        in_specs=[
            pl.BlockSpec((BR, 2), lambda i: (i, 0)),
            pl.BlockSpec((BR, D), lambda i: (i, 0)),
        ],
        out_specs=pl.BlockSpec((BR, D), lambda i: (i, 0)),
        out_shape=jax.ShapeDtypeStruct((N, D), jnp.float32),
    )(degp, mm)


def _tc_b(degp, s, xs1, b1, W2):
    return pl.pallas_call(
        _tc_b_body,
        grid=(N // BR,),
        in_specs=[
            pl.BlockSpec((BR, 2), lambda i: (i, 0)),
            pl.BlockSpec((NC, BR, D), lambda i: (0, i, 0)),
            pl.BlockSpec((BR, D), lambda i: (i, 0)),
            pl.BlockSpec((D,), lambda i: (0,)),
            pl.BlockSpec((D, D), lambda i: (0, 0)),
        ],
        out_specs=pl.BlockSpec((BR, D), lambda i: (i, 0)),
        out_shape=jax.ShapeDtypeStruct((N, D), jnp.float32),
    )(degp, s, xs1, b1, W2)


def _tc_c(degp, q, xs2, b2):
    return pl.pallas_call(
        _tc_c_body,
        grid=(N // BR,),
        in_specs=[
            pl.BlockSpec((BR, 2), lambda i: (i, 0)),
            pl.BlockSpec((NC, BR, D), lambda i: (0, i, 0)),
            pl.BlockSpec((BR, D), lambda i: (i, 0)),
            pl.BlockSpec((D,), lambda i: (0,)),
        ],
        out_specs=pl.BlockSpec((BR, D), lambda i: (i, 0)),
        out_shape=jax.ShapeDtypeStruct((N, D), jnp.float32),
    )(degp, q, xs2, b2)


def kernel(x, edge_index, W1, b1, W2, b2):
    src = edge_index[0].astype(jnp.int32)
    dst = edge_index[1].astype(jnp.int32)
    pad = E_PAD - E
    pad_i = jnp.arange(pad, dtype=jnp.int32)
    src_p = jnp.concatenate([src, pad_i % N])
    dst_p = jnp.concatenate([dst, DUMMY + pad_i % (ACC_ROWS - N)])
    pk4 = ((dst_p << SHIFT) | src_p).reshape(NW, NWIN, NB, CHUNK)
    zeros_deg = jnp.zeros((ZROWS,), jnp.float32)
    zeros_rows = jnp.zeros((ZROWS, D), jnp.float32)

    mm1 = _tc_mm(x, W1)
    degp2 = _deg_kernel(pk4, zeros_deg)
    degp = degp2[:, :N].T

    xs1 = _tc_scale(degp, mm1)
    s = _scatter_kernel(xs1, pk4, zeros_rows)
    xs2 = _tc_b(degp, s, xs1, b1, W2)
    q = _scatter_kernel(xs2, pk4, zeros_rows)
    return _tc_c(degp, q, xs2, b2)


# speedup vs baseline: 1.0242x; 1.0242x over previous
import jax
import jax.numpy as jnp
from jax import lax
from jax.experimental import pallas as pl
from jax.experimental.pallas import tpu as pltpu
from jax.experimental.pallas import tpu_sc as plsc

N = 10000
D = 128
E = 320000

NC = 2
NS = 16
NW = NC * NS

CHUNK = 80
EPT = 10240
CPT = EPT // CHUNK
E_PAD = NW * EPT
DUMMY = N
ACC_ROWS = NW * 320
ZROWS = ACC_ROWS // NS
NB = 4
NWIN = CPT // NB
SHIFT = 14
MASK = (1 << SHIFT) - 1

_mesh = plsc.VectorSubcoreMesh(
    core_axis_name="c", subcore_axis_name="s", num_cores=NC, num_subcores=NS
)


def _unpack(pkb, b, src_b, dst_b):
    for k in range(CHUNK // 16):
        v = pkb[b, pl.ds(k * 16, 16)]
        src_b[pl.ds(k * 16, 16)] = jnp.bitwise_and(v, MASK)
        dst_b[pl.ds(k * 16, 16)] = lax.shift_right_logical(v, SHIFT)


def _deg_unpack_dst(pkb, b, dst_b):
    for k in range(CHUNK // 16):
        v = pkb[b, pl.ds(k * 16, 16)]
        dst_b[pl.ds(k * 16, 16)] = lax.shift_right_logical(v, SHIFT)


def _deg_body(pk_hbm, zeros_hbm, out_hbm,
              pkb0, pkb1, db0, db1, db2, db3, ones_v,
              p0, p1, s0, s1, s2, s3, deg_sh):
    pkb = (pkb0, pkb1)
    psem = (p0, p1)
    dbuf = (db0, db1, db2, db3)
    ssem = (s0, s1, s2, s3)
    cid = lax.axis_index("c")
    sid = lax.axis_index("s")
    wid = cid * NS + sid

    def pk_fetch(w, half):
        pltpu.async_copy(pk_hbm.at[wid, w], pkb[half], psem[half])

    def pk_wait(half):
        pltpu.make_async_copy(pk_hbm.at[wid, 0], pkb[half], psem[half]).wait()

    def scatter(b):
        pltpu.async_copy(ones_v, deg_sh.at[dbuf[b]], ssem[b], add=True)

    def scatter_wait(b):
        pltpu.make_async_copy(ones_v, deg_sh.at[dbuf[b]], ssem[b]).wait()

    pltpu.sync_copy(zeros_hbm, deg_sh.at[pl.ds(sid * ZROWS, ZROWS)])
    pk_fetch(0, 0)
    pk_fetch(1, 1)
    for k in range(CHUNK // 16):
        ones_v[pl.ds(k * 16, 16)] = jnp.ones((16,), jnp.float32)
    plsc.subcore_barrier()
    pk_wait(0)
    for b in range(NB):
        _deg_unpack_dst(pkb0, b, dbuf[b])
        scatter(b)

    def macro(i, carry):
        w2 = 2 * i + 2
        pk_fetch(w2, 0)
        pk_wait(1)
        for b in range(NB):
            scatter_wait(b)
            _deg_unpack_dst(pkb1, b, dbuf[b])
            scatter(b)
        pk_fetch(w2 + 1, 1)
        pk_wait(0)
        for b in range(NB):
            scatter_wait(b)
            _deg_unpack_dst(pkb0, b, dbuf[b])
            scatter(b)
        return carry

    lax.fori_loop(0, NWIN // 2 - 1, macro, 0)
    pk_wait(1)
    for b in range(NB):
        scatter_wait(b)
        _deg_unpack_dst(pkb1, b, dbuf[b])
        scatter(b)
    for b in range(NB):
        scatter_wait(b)
    plsc.subcore_barrier()
    pltpu.sync_copy(
        deg_sh.at[pl.ds(sid * ZROWS, ZROWS)],
        out_hbm.at[cid, pl.ds(sid * ZROWS, ZROWS)],
    )


_deg_kernel = pl.kernel(
    _deg_body,
    out_type=jax.ShapeDtypeStruct((NC, ACC_ROWS), jnp.float32),
    mesh=_mesh,
    scratch_types=(
        [pltpu.VMEM((NB, CHUNK), jnp.int32)] * 2
        + [pltpu.VMEM((CHUNK,), jnp.int32)] * NB
        + [pltpu.VMEM((CHUNK,), jnp.float32)]
        + [pltpu.SemaphoreType.DMA] * (2 + NB)
        + [pltpu.VMEM_SHARED((ACC_ROWS,), jnp.float32)]
    ),
)


def _scatter_body(xs_hbm, pk_hbm, zeros_hbm, out_hbm,
                  pkb0, pkb1, sb0, sb1, sb2, sb3, db0, db1, db2, db3,
                  r0, r1, r2, r3,
                  p0, p1, g0, g1, g2, g3, s0, s1, s2, s3, zsem, acc_sh):
    pkb = (pkb0, pkb1)
    psem = (p0, p1)
    sbuf = (sb0, sb1, sb2, sb3)
    dbuf = (db0, db1, db2, db3)
    rows = (r0, r1, r2, r3)
    gsem = (g0, g1, g2, g3)
    ssem = (s0, s1, s2, s3)
    cid = lax.axis_index("c")
    sid = lax.axis_index("s")
    wid = cid * NS + sid

    def pk_fetch(w, half):
        pltpu.async_copy(pk_hbm.at[wid, w], pkb[half], psem[half])

    def pk_wait(half):
        pltpu.make_async_copy(pk_hbm.at[wid, 0], pkb[half], psem[half]).wait()

    def gather(b):
        pltpu.async_copy(xs_hbm.at[sbuf[b]], rows[b], gsem[b])

    def gather_wait(b):
        pltpu.make_async_copy(xs_hbm.at[sbuf[b]], rows[b], gsem[b]).wait()

    def scatter(b):
        pltpu.async_copy(rows[b], acc_sh.at[dbuf[b]], ssem[b], add=True)

    def scatter_wait(b):
        pltpu.make_async_copy(rows[b], acc_sh.at[dbuf[b]], ssem[b]).wait()

    zcopy = pltpu.make_async_copy(
        zeros_hbm, acc_sh.at[pl.ds(sid * ZROWS, ZROWS)], zsem
    )
    zcopy.start()
    pk_fetch(0, 0)
    pk_fetch(1, 1)
    pk_wait(0)
    for b in range(NB):
        _unpack(pkb0, b, sbuf[b], dbuf[b])
        gather(b)
    zcopy.wait()
    plsc.subcore_barrier()

    def macro(i, carry):
        w2 = 2 * i + 2
        pk_fetch(w2, 0)
        for b in range(NB):
            gather_wait(b)
            scatter(b)
        pk_wait(1)
        for b in range(NB):
            scatter_wait(b)
            _unpack(pkb1, b, sbuf[b], dbuf[b])
            gather(b)
        pk_fetch(w2 + 1, 1)
        for b in range(NB):
            gather_wait(b)
            scatter(b)
        pk_wait(0)
        for b in range(NB):
            scatter_wait(b)
            _unpack(pkb0, b, sbuf[b], dbuf[b])
            gather(b)
        return carry

    lax.fori_loop(0, NWIN // 2 - 1, macro, 0)
    for b in range(NB):
        gather_wait(b)
        scatter(b)
    pk_wait(1)
    for b in range(NB):
        scatter_wait(b)
        _unpack(pkb1, b, sbuf[b], dbuf[b])
        gather(b)
    for b in range(NB):
        gather_wait(b)
        scatter(b)
    for b in range(NB):
        scatter_wait(b)
    plsc.subcore_barrier()
    pltpu.sync_copy(
        acc_sh.at[pl.ds(sid * ZROWS, ZROWS)],
        out_hbm.at[cid, pl.ds(sid * ZROWS, ZROWS)],
    )


_scatter_kernel = pl.kernel(
    _scatter_body,
    out_type=jax.ShapeDtypeStruct((NC, ACC_ROWS, D), jnp.float32),
    mesh=_mesh,
    scratch_types=(
        [pltpu.VMEM((NB, CHUNK), jnp.int32)] * 2
        + [pltpu.VMEM((CHUNK,), jnp.int32)] * (2 * NB)
        + [pltpu.VMEM((CHUNK, D), jnp.float32)] * NB
        + [pltpu.SemaphoreType.DMA] * (3 + 2 * NB)
        + [pltpu.VMEM_SHARED((ACC_ROWS, D), jnp.float32)]
    ),
)


BR = 2000


def _dinv(degp_ref):
    deg = degp_ref[:, 0:1] + degp_ref[:, 1:2] + 1.0
    return lax.rsqrt(deg)


def _tc_a_body(degp_ref, x_ref, w_ref, xs_ref):
    dv = _dinv(degp_ref)
    xs_ref[...] = (
        jnp.dot(x_ref[...], w_ref[...], preferred_element_type=jnp.float32) * dv
    )


def _tc_b_body(degp_ref, s_ref, xs_ref, b_ref, w_ref, out_ref):
    dv = _dinv(degp_ref)
    h = s_ref[0] + s_ref[1] + xs_ref[...]
    h = jnp.maximum(dv * h + b_ref[...], 0.0)
    out_ref[...] = (
        jnp.dot(h, w_ref[...], preferred_element_type=jnp.float32) * dv
    )


def _tc_c_body(degp_ref, q_ref, xs_ref, b_ref, out_ref):
    dv = _dinv(degp_ref)
    out_ref[...] = dv * (q_ref[0] + q_ref[1] + xs_ref[...]) + b_ref[...]


def _tc_a(degp, x, W1):
    return pl.pallas_call(
        _tc_a_body,
        grid=(N // BR,),
        in_specs=[
            pl.BlockSpec((BR, 2), lambda i: (i, 0)),
            pl.BlockSpec((BR, D), lambda i: (i, 0)),
            pl.BlockSpec((D, D), lambda i: (0, 0)),
        ],
        out_specs=pl.BlockSpec((BR, D), lambda i: (i, 0)),
        out_shape=jax.ShapeDtypeStruct((N, D), jnp.float32),
    )(degp, x, W1)


def _tc_b(degp, s, xs1, b1, W2):
    return pl.pallas_call(
        _tc_b_body,
        grid=(N // BR,),
        in_specs=[
            pl.BlockSpec((BR, 2), lambda i: (i, 0)),
            pl.BlockSpec((NC, BR, D), lambda i: (0, i, 0)),
            pl.BlockSpec((BR, D), lambda i: (i, 0)),
            pl.BlockSpec((D,), lambda i: (0,)),
            pl.BlockSpec((D, D), lambda i: (0, 0)),
        ],
        out_specs=pl.BlockSpec((BR, D), lambda i: (i, 0)),
        out_shape=jax.ShapeDtypeStruct((N, D), jnp.float32),
    )(degp, s, xs1, b1, W2)


def _tc_c(degp, q, xs2, b2):
    return pl.pallas_call(
        _tc_c_body,
        grid=(N // BR,),
        in_specs=[
            pl.BlockSpec((BR, 2), lambda i: (i, 0)),
            pl.BlockSpec((NC, BR, D), lambda i: (0, i, 0)),
            pl.BlockSpec((BR, D), lambda i: (i, 0)),
            pl.BlockSpec((D,), lambda i: (0,)),
        ],
        out_specs=pl.BlockSpec((BR, D), lambda i: (i, 0)),
        out_shape=jax.ShapeDtypeStruct((N, D), jnp.float32),
    )(degp, q, xs2, b2)


def kernel(x, edge_index, W1, b1, W2, b2):
    src = edge_index[0].astype(jnp.int32)
    dst = edge_index[1].astype(jnp.int32)
    pad = E_PAD - E
    pad_i = jnp.arange(pad, dtype=jnp.int32)
    src_p = jnp.concatenate([src, pad_i % N])
    dst_p = jnp.concatenate([dst, DUMMY + pad_i % (ACC_ROWS - N)])
    pk4 = ((dst_p << SHIFT) | src_p).reshape(NW, NWIN, NB, CHUNK)
    zeros_deg = jnp.zeros((ZROWS,), jnp.float32)
    zeros_rows = jnp.zeros((ZROWS, D), jnp.float32)

    degp2 = _deg_kernel(pk4, zeros_deg)
    degp = degp2[:, :N].T

    xs1 = _tc_a(degp, x, W1)
    s = _scatter_kernel(xs1, pk4, zeros_rows)
    xs2 = _tc_b(degp, s, xs1, b1, W2)
    q = _scatter_kernel(xs2, pk4, zeros_rows)
    return _tc_c(degp, q, xs2, b2)


# speedup vs baseline: 1.0249x; 1.0007x over previous
import jax
import jax.numpy as jnp
from jax import lax
from jax.experimental import pallas as pl
from jax.experimental.pallas import tpu as pltpu
from jax.experimental.pallas import tpu_sc as plsc

N = 10000
D = 128
E = 320000

NC = 2
NS = 16
NW = NC * NS

CHUNK = 64
EPT = 10240
CPT = EPT // CHUNK
E_PAD = NW * EPT
DUMMY = N
ACC_ROWS = NW * 320
ZROWS = ACC_ROWS // NS
NB = 5
NWIN = CPT // NB
SHIFT = 14
MASK = (1 << SHIFT) - 1

_mesh = plsc.VectorSubcoreMesh(
    core_axis_name="c", subcore_axis_name="s", num_cores=NC, num_subcores=NS
)


def _unpack(pkb, b, src_b, dst_b):
    for k in range(CHUNK // 16):
        v = pkb[b, pl.ds(k * 16, 16)]
        src_b[pl.ds(k * 16, 16)] = jnp.bitwise_and(v, MASK)
        dst_b[pl.ds(k * 16, 16)] = lax.shift_right_logical(v, SHIFT)


def _deg_unpack_dst(pkb, b, dst_b):
    for k in range(CHUNK // 16):
        v = pkb[b, pl.ds(k * 16, 16)]
        dst_b[pl.ds(k * 16, 16)] = lax.shift_right_logical(v, SHIFT)


def _deg_body(pk_hbm, zeros_hbm, out_hbm,
              pkb0, pkb1, db0, db1, db2, db3, db4, ones_v,
              p0, p1, s0, s1, s2, s3, s4, deg_sh):
    pkb = (pkb0, pkb1)
    psem = (p0, p1)
    dbuf = (db0, db1, db2, db3, db4)
    ssem = (s0, s1, s2, s3, s4)
    cid = lax.axis_index("c")
    sid = lax.axis_index("s")
    wid = cid * NS + sid

    def pk_fetch(w, half):
        pltpu.async_copy(pk_hbm.at[wid, w], pkb[half], psem[half])

    def pk_wait(half):
        pltpu.make_async_copy(pk_hbm.at[wid, 0], pkb[half], psem[half]).wait()

    def scatter(b):
        pltpu.async_copy(ones_v, deg_sh.at[dbuf[b]], ssem[b], add=True)

    def scatter_wait(b):
        pltpu.make_async_copy(ones_v, deg_sh.at[dbuf[b]], ssem[b]).wait()

    pltpu.sync_copy(zeros_hbm, deg_sh.at[pl.ds(sid * ZROWS, ZROWS)])
    pk_fetch(0, 0)
    pk_fetch(1, 1)
    for k in range(CHUNK // 16):
        ones_v[pl.ds(k * 16, 16)] = jnp.ones((16,), jnp.float32)
    plsc.subcore_barrier()
    pk_wait(0)
    for b in range(NB):
        _deg_unpack_dst(pkb0, b, dbuf[b])
        scatter(b)

    def macro(i, carry):
        w2 = 2 * i + 2
        pk_fetch(w2, 0)
        pk_wait(1)
        for b in range(NB):
            scatter_wait(b)
            _deg_unpack_dst(pkb1, b, dbuf[b])
            scatter(b)
        pk_fetch(w2 + 1, 1)
        pk_wait(0)
        for b in range(NB):
            scatter_wait(b)
            _deg_unpack_dst(pkb0, b, dbuf[b])
            scatter(b)
        return carry

    lax.fori_loop(0, NWIN // 2 - 1, macro, 0)
    pk_wait(1)
    for b in range(NB):
        scatter_wait(b)
        _deg_unpack_dst(pkb1, b, dbuf[b])
        scatter(b)
    for b in range(NB):
        scatter_wait(b)
    plsc.subcore_barrier()
    pltpu.sync_copy(
        deg_sh.at[pl.ds(sid * ZROWS, ZROWS)],
        out_hbm.at[cid, pl.ds(sid * ZROWS, ZROWS)],
    )


_deg_kernel = pl.kernel(
    _deg_body,
    out_type=jax.ShapeDtypeStruct((NC, ACC_ROWS), jnp.float32),
    mesh=_mesh,
    scratch_types=(
        [pltpu.VMEM((NB, CHUNK), jnp.int32)] * 2
        + [pltpu.VMEM((CHUNK,), jnp.int32)] * NB
        + [pltpu.VMEM((CHUNK,), jnp.float32)]
        + [pltpu.SemaphoreType.DMA] * (2 + NB)
        + [pltpu.VMEM_SHARED((ACC_ROWS,), jnp.float32)]
    ),
)


def _scatter_body(xs_hbm, pk_hbm, zeros_hbm, out_hbm,
                  pkb0, pkb1, sb0, sb1, sb2, sb3, sb4, db0, db1, db2, db3, db4,
                  r0, r1, r2, r3, r4,
                  p0, p1, g0, g1, g2, g3, g4, s0, s1, s2, s3, s4, zsem, acc_sh):
    pkb = (pkb0, pkb1)
    psem = (p0, p1)
    sbuf = (sb0, sb1, sb2, sb3, sb4)
    dbuf = (db0, db1, db2, db3, db4)
    rows = (r0, r1, r2, r3, r4)
    gsem = (g0, g1, g2, g3, g4)
    ssem = (s0, s1, s2, s3, s4)
    cid = lax.axis_index("c")
    sid = lax.axis_index("s")
    wid = cid * NS + sid

    def pk_fetch(w, half):
        pltpu.async_copy(pk_hbm.at[wid, w], pkb[half], psem[half])

    def pk_wait(half):
        pltpu.make_async_copy(pk_hbm.at[wid, 0], pkb[half], psem[half]).wait()

    def gather(b):
        pltpu.async_copy(xs_hbm.at[sbuf[b]], rows[b], gsem[b])

    def gather_wait(b):
        pltpu.make_async_copy(xs_hbm.at[sbuf[b]], rows[b], gsem[b]).wait()

    def scatter(b):
        pltpu.async_copy(rows[b], acc_sh.at[dbuf[b]], ssem[b], add=True)

    def scatter_wait(b):
        pltpu.make_async_copy(rows[b], acc_sh.at[dbuf[b]], ssem[b]).wait()

    zcopy = pltpu.make_async_copy(
        zeros_hbm, acc_sh.at[pl.ds(sid * ZROWS, ZROWS)], zsem
    )
    zcopy.start()
    pk_fetch(0, 0)
    pk_fetch(1, 1)
    pk_wait(0)
    for b in range(NB):
        _unpack(pkb0, b, sbuf[b], dbuf[b])
        gather(b)
    zcopy.wait()
    plsc.subcore_barrier()

    def macro(i, carry):
        w2 = 2 * i + 2
        pk_fetch(w2, 0)
        for b in range(NB):
            gather_wait(b)
            scatter(b)
        pk_wait(1)
        for b in range(NB):
            scatter_wait(b)
            _unpack(pkb1, b, sbuf[b], dbuf[b])
            gather(b)
        pk_fetch(w2 + 1, 1)
        for b in range(NB):
            gather_wait(b)
            scatter(b)
        pk_wait(0)
        for b in range(NB):
            scatter_wait(b)
            _unpack(pkb0, b, sbuf[b], dbuf[b])
            gather(b)
        return carry

    lax.fori_loop(0, NWIN // 2 - 1, macro, 0)
    for b in range(NB):
        gather_wait(b)
        scatter(b)
    pk_wait(1)
    for b in range(NB):
        scatter_wait(b)
        _unpack(pkb1, b, sbuf[b], dbuf[b])
        gather(b)
    for b in range(NB):
        gather_wait(b)
        scatter(b)
    for b in range(NB):
        scatter_wait(b)
    plsc.subcore_barrier()
    pltpu.sync_copy(
        acc_sh.at[pl.ds(sid * ZROWS, ZROWS)],
        out_hbm.at[cid, pl.ds(sid * ZROWS, ZROWS)],
    )


_scatter_kernel = pl.kernel(
    _scatter_body,
    out_type=jax.ShapeDtypeStruct((NC, ACC_ROWS, D), jnp.float32),
    mesh=_mesh,
    scratch_types=(
        [pltpu.VMEM((NB, CHUNK), jnp.int32)] * 2
        + [pltpu.VMEM((CHUNK,), jnp.int32)] * (2 * NB)
        + [pltpu.VMEM((CHUNK, D), jnp.float32)] * NB
        + [pltpu.SemaphoreType.DMA] * (3 + 2 * NB)
        + [pltpu.VMEM_SHARED((ACC_ROWS, D), jnp.float32)]
    ),
)


BR = 2000


def _dinv(degp_ref):
    deg = degp_ref[:, 0:1] + degp_ref[:, 1:2] + 1.0
    return lax.rsqrt(deg)


def _tc_a_body(degp_ref, x_ref, w_ref, xs_ref):
    dv = _dinv(degp_ref)
    xs_ref[...] = (
        jnp.dot(x_ref[...], w_ref[...], preferred_element_type=jnp.float32) * dv
    )


def _tc_b_body(degp_ref, s_ref, xs_ref, b_ref, w_ref, out_ref):
    dv = _dinv(degp_ref)
    h = s_ref[0] + s_ref[1] + xs_ref[...]
    h = jnp.maximum(dv * h + b_ref[...], 0.0)
    out_ref[...] = (
        jnp.dot(h, w_ref[...], preferred_element_type=jnp.float32) * dv
    )


def _tc_c_body(degp_ref, q_ref, xs_ref, b_ref, out_ref):
    dv = _dinv(degp_ref)
    out_ref[...] = dv * (q_ref[0] + q_ref[1] + xs_ref[...]) + b_ref[...]


def _tc_a(degp, x, W1):
    return pl.pallas_call(
        _tc_a_body,
        grid=(N // BR,),
        in_specs=[
            pl.BlockSpec((BR, 2), lambda i: (i, 0)),
            pl.BlockSpec((BR, D), lambda i: (i, 0)),
            pl.BlockSpec((D, D), lambda i: (0, 0)),
        ],
        out_specs=pl.BlockSpec((BR, D), lambda i: (i, 0)),
        out_shape=jax.ShapeDtypeStruct((N, D), jnp.float32),
    )(degp, x, W1)


def _tc_b(degp, s, xs1, b1, W2):
    return pl.pallas_call(
        _tc_b_body,
        grid=(N // BR,),
        in_specs=[
            pl.BlockSpec((BR, 2), lambda i: (i, 0)),
            pl.BlockSpec((NC, BR, D), lambda i: (0, i, 0)),
            pl.BlockSpec((BR, D), lambda i: (i, 0)),
            pl.BlockSpec((D,), lambda i: (0,)),
            pl.BlockSpec((D, D), lambda i: (0, 0)),
        ],
        out_specs=pl.BlockSpec((BR, D), lambda i: (i, 0)),
        out_shape=jax.ShapeDtypeStruct((N, D), jnp.float32),
    )(degp, s, xs1, b1, W2)


def _tc_c(degp, q, xs2, b2):
    return pl.pallas_call(
        _tc_c_body,
        grid=(N // BR,),
        in_specs=[
            pl.BlockSpec((BR, 2), lambda i: (i, 0)),
            pl.BlockSpec((NC, BR, D), lambda i: (0, i, 0)),
            pl.BlockSpec((BR, D), lambda i: (i, 0)),
            pl.BlockSpec((D,), lambda i: (0,)),
        ],
        out_specs=pl.BlockSpec((BR, D), lambda i: (i, 0)),
        out_shape=jax.ShapeDtypeStruct((N, D), jnp.float32),
    )(degp, q, xs2, b2)


def kernel(x, edge_index, W1, b1, W2, b2):
    src = edge_index[0].astype(jnp.int32)
    dst = edge_index[1].astype(jnp.int32)
    pad = E_PAD - E
    pad_i = jnp.arange(pad, dtype=jnp.int32)
    src_p = jnp.concatenate([src, pad_i % N])
    dst_p = jnp.concatenate([dst, DUMMY + pad_i % (ACC_ROWS - N)])
    pk4 = ((dst_p << SHIFT) | src_p).reshape(NW, NWIN, NB, CHUNK)
    zeros_deg = jnp.zeros((ZROWS,), jnp.float32)
    zeros_rows = jnp.zeros((ZROWS, D), jnp.float32)

    degp2 = _deg_kernel(pk4, zeros_deg)
    degp = degp2[:, :N].T

    xs1 = _tc_a(degp, x, W1)
    s = _scatter_kernel(xs1, pk4, zeros_rows)
    xs2 = _tc_b(degp, s, xs1, b1, W2)
    q = _scatter_kernel(xs2, pk4, zeros_rows)
    return _tc_c(degp, q, xs2, b2)
